# Initial kernel scaffold; baseline (speedup 1.0000x reference)
import jax
import jax.numpy as jnp
from jax.experimental import pallas as pl


def kernel(flat_tokens, cu_seqlens, lookup_table):
    raise NotImplementedError("write your pallas kernel here")


# speedup vs baseline: 10.9139x; 10.9139x over previous
import jax
import jax.numpy as jnp
from jax import lax
from jax.experimental import pallas as pl
from jax.experimental.pallas import tpu as pltpu
from jax.experimental.pallas import tpu_sc as plsc

_B = 16
_TOTAL = 32768
_L = 2048
_LANES = 16
_NC = 2
_NS = 16
_NW = _NC * _NS
_CHUNK = (_B * _L) // _NW
_STAGE = _CHUNK + 8
_NSTREAM = 8
_IW = _CHUNK // _NSTREAM


def _tok_body(flat_hbm, cu_hbm, table_hbm, out_hbm, cu_v, ft_v, tok_v, val_v, sem):
    wid = lax.axis_index("s") * _NC + lax.axis_index("c")
    b = wid // 2
    h = wid % 2
    l0 = h * _CHUNK

    pltpu.sync_copy(cu_hbm, cu_v)

    lane = lax.iota(jnp.int32, _LANES)
    b_splat = jnp.full((_LANES,), b, jnp.int32)
    start = jnp.max(plsc.load_gather(cu_v, [b_splat]))
    end_splat = plsc.load_gather(cu_v, [b_splat + 1])

    off_raw = start + l0
    off = pl.multiple_of(jnp.minimum(off_raw & -8, _TOTAL - _STAGE), 8)
    pltpu.sync_copy(flat_hbm.at[pl.ds(off, _STAGE)], ft_v)

    base = off_raw - off

    def prep(j, carry):
        lpos = j * _LANES + lane
        lidx = jnp.clip(base + lpos, 0, _STAGE - 1)
        tok = plsc.load_gather(ft_v, [lidx])
        plsc.store_scatter(tok_v, [lpos], tok)
        return carry

    lax.fori_loop(0, _CHUNK // _LANES, prep, 0)

    copies = [
        pltpu.async_copy(
            table_hbm.at[tok_v.at[pl.ds(r * _IW, _IW)]],
            val_v.at[pl.ds(r * _IW, _IW)],
            sem,
        )
        for r in range(_NSTREAM)
    ]
    for c in copies:
        c.wait()

    def maskfix(j, carry):
        lpos = j * _LANES + lane
        val = plsc.load_gather(val_v, [lpos])
        keep = (off_raw + lpos) < end_splat
        plsc.store_scatter(val_v, [lpos], jnp.where(keep, val, 0.0))
        return carry

    lax.fori_loop(0, _CHUNK // _LANES, maskfix, 0)

    pltpu.sync_copy(val_v, out_hbm.at[b, h])


def kernel(flat_tokens, cu_seqlens, lookup_table):
    cu_pad = jnp.pad(cu_seqlens.astype(jnp.int32), (0, 32 - _B - 1))
    mesh = plsc.VectorSubcoreMesh(core_axis_name="c", subcore_axis_name="s")
    out = pl.kernel(
        _tok_body,
        out_type=jax.ShapeDtypeStruct((_B, 2, _CHUNK), jnp.float32),
        mesh=mesh,
        compiler_params=pltpu.CompilerParams(needs_layout_passes=False),
        scratch_types=[
            pltpu.VMEM((32,), jnp.int32),
            pltpu.VMEM((_STAGE,), jnp.int32),
            pltpu.VMEM((_CHUNK,), jnp.int32),
            pltpu.VMEM((_CHUNK,), jnp.float32),
            pltpu.SemaphoreType.DMA,
        ],
    )(flat_tokens, cu_pad, lookup_table)
    return out.reshape(_B, _L)


# speedup vs baseline: 10.9551x; 1.0038x over previous
import jax
import jax.numpy as jnp
from jax import lax
from jax.experimental import pallas as pl
from jax.experimental.pallas import tpu as pltpu
from jax.experimental.pallas import tpu_sc as plsc

_B = 16
_TOTAL = 32768
_L = 2048
_LANES = 16
_NC = 2
_NS = 16
_NW = _NC * _NS
_CHUNK = (_B * _L) // _NW
_STAGE = _CHUNK + 8
_NSTREAM = 8
_IW = _CHUNK // _NSTREAM


def _tok_body(flat_hbm, cu_hbm, table_hbm, out_hbm, cu_v, ft_v, tok_v, val_v, sem):
    wid = lax.axis_index("s") * _NC + lax.axis_index("c")
    b = wid // 2
    h = wid % 2
    l0 = h * _CHUNK

    pltpu.sync_copy(cu_hbm, cu_v)

    lane = lax.iota(jnp.int32, _LANES)
    b_splat = jnp.full((_LANES,), b, jnp.int32)
    start = jnp.max(plsc.load_gather(cu_v, [b_splat]))
    end_splat = plsc.load_gather(cu_v, [b_splat + 1])

    off_raw = start + l0
    off = pl.multiple_of(jnp.minimum(off_raw & -8, _TOTAL - _STAGE), 8)
    pltpu.sync_copy(flat_hbm.at[pl.ds(off, _STAGE)], ft_v)

    base = off_raw - off

    copies = []
    for r in range(_NSTREAM):

        @plsc.parallel_loop(r * (_IW // _LANES), (r + 1) * (_IW // _LANES),
                            unroll=4)
        def prep(j):
            lpos = j * _LANES + lane
            lidx = jnp.clip(base + lpos, 0, _STAGE - 1)
            tok = plsc.load_gather(ft_v, [lidx])
            plsc.store_scatter(tok_v, [lpos], tok)

        copies.append(
            pltpu.async_copy(
                table_hbm.at[tok_v.at[pl.ds(r * _IW, _IW)]],
                val_v.at[pl.ds(r * _IW, _IW)],
                sem,
            )
        )
    for c in copies:
        c.wait()

    @plsc.parallel_loop(0, _CHUNK // _LANES, unroll=4)
    def maskfix(j):
        lpos = j * _LANES + lane
        val = plsc.load_gather(val_v, [lpos])
        keep = (off_raw + lpos) < end_splat
        plsc.store_scatter(val_v, [lpos], jnp.where(keep, val, 0.0))

    pltpu.sync_copy(val_v, out_hbm.at[b, h])


def kernel(flat_tokens, cu_seqlens, lookup_table):
    mesh = plsc.VectorSubcoreMesh(core_axis_name="c", subcore_axis_name="s")
    out = pl.kernel(
        _tok_body,
        out_type=jax.ShapeDtypeStruct((_B, 2, _CHUNK), jnp.float32),
        mesh=mesh,
        compiler_params=pltpu.CompilerParams(needs_layout_passes=False),
        scratch_types=[
            pltpu.VMEM((_B + 1,), jnp.int32),
            pltpu.VMEM((_STAGE,), jnp.int32),
            pltpu.VMEM((_CHUNK,), jnp.int32),
            pltpu.VMEM((_CHUNK,), jnp.float32),
            pltpu.SemaphoreType.DMA,
        ],
    )(flat_tokens, cu_seqlens.astype(jnp.int32), lookup_table)
    return out.reshape(_B, _L)


# speedup vs baseline: 10.9899x; 1.0032x over previous
import jax
import jax.numpy as jnp
from jax import lax
from jax.experimental import pallas as pl
from jax.experimental.pallas import tpu as pltpu
from jax.experimental.pallas import tpu_sc as plsc

_B = 16
_TOTAL = 32768
_L = 2048
_LANES = 16
_NC = 2
_NS = 16
_NW = _NC * _NS
_CHUNK = (_B * _L) // _NW
_STAGE = _CHUNK + 8
_NSTREAM = 8
_IW = _CHUNK // _NSTREAM
_NCH = _CHUNK // _LANES


def _tok_body(flat_hbm, cu_hbm, table_hbm, out_hbm, cu_v, ft_v, tok_v, val_v, sem):
    wid = lax.axis_index("s") * _NC + lax.axis_index("c")
    b = wid // 2
    h = wid % 2
    l0 = h * _CHUNK

    pltpu.sync_copy(cu_hbm, cu_v)

    lane = lax.iota(jnp.int32, _LANES)
    b_splat = jnp.full((_LANES,), b, jnp.int32)
    start = jnp.max(plsc.load_gather(cu_v, [b_splat]))
    end_splat = plsc.load_gather(cu_v, [b_splat + 1])
    end = jnp.max(end_splat)

    off_raw = start + l0
    n_valid = jnp.clip(end - off_raw, 0, _CHUNK)

    off = pl.multiple_of(jnp.minimum(off_raw & -8, _TOTAL - _STAGE), 8)
    base = off_raw - off

    @pl.when(n_valid > 0)
    def _stage():
        pltpu.sync_copy(flat_hbm.at[pl.ds(off, _STAGE)], ft_v)

    copies = []
    for r in range(_NSTREAM):
        fire = n_valid > r * _IW

        @pl.when(fire)
        def _prep_and_fire():
            @plsc.parallel_loop(r * (_IW // _LANES), (r + 1) * (_IW // _LANES),
                                unroll=4)
            def prep(j):
                lpos = j * _LANES + lane
                lidx = jnp.clip(base + lpos, 0, _STAGE - 1)
                tok = plsc.load_gather(ft_v, [lidx])
                plsc.store_scatter(tok_v, [lpos], tok)

            pltpu.async_copy(
                table_hbm.at[tok_v.at[pl.ds(r * _IW, _IW)]],
                val_v.at[pl.ds(r * _IW, _IW)],
                sem,
            )

        copies.append(fire)
    for r, fire in enumerate(copies):
        @pl.when(fire)
        def _drain():
            pltpu.make_async_copy(
                table_hbm.at[tok_v.at[pl.ds(r * _IW, _IW)]],
                val_v.at[pl.ds(r * _IW, _IW)],
                sem,
            ).wait()

    jz = n_valid // _LANES

    @pl.when(jz < _NCH)
    def _mixed():
        lpos = jz * _LANES + lane
        val = plsc.load_gather(val_v, [lpos])
        keep = (off_raw + lpos) < end_splat
        plsc.store_scatter(val_v, [lpos], jnp.where(keep, val, 0.0))

    zeros = jnp.zeros((_LANES,), jnp.float32)

    def zfill(j, c):
        plsc.store_scatter(val_v, [j * _LANES + lane], zeros)
        return c

    lax.fori_loop(jz + 1, _NCH, zfill, 0)

    pltpu.sync_copy(val_v, out_hbm.at[b, h])


def kernel(flat_tokens, cu_seqlens, lookup_table):
    mesh = plsc.VectorSubcoreMesh(core_axis_name="c", subcore_axis_name="s")
    out = pl.kernel(
        _tok_body,
        out_type=jax.ShapeDtypeStruct((_B, 2, _CHUNK), jnp.float32),
        mesh=mesh,
        compiler_params=pltpu.CompilerParams(needs_layout_passes=False),
        scratch_types=[
            pltpu.VMEM((_B + 1,), jnp.int32),
            pltpu.VMEM((_STAGE,), jnp.int32),
            pltpu.VMEM((_CHUNK,), jnp.int32),
            pltpu.VMEM((_CHUNK,), jnp.float32),
            pltpu.SemaphoreType.DMA,
        ],
    )(flat_tokens, cu_seqlens.astype(jnp.int32), lookup_table)
    return out.reshape(_B, _L)


# speedup vs baseline: 11.2204x; 1.0210x over previous
import jax
import jax.numpy as jnp
from jax import lax
from jax.experimental import pallas as pl
from jax.experimental.pallas import tpu as pltpu
from jax.experimental.pallas import tpu_sc as plsc

_B = 16
_TOTAL = 32768
_L = 2048
_LANES = 16
_NC = 2
_NS = 16
_NW = _NC * _NS
_CHUNK = (_B * _L) // _NW
_STAGE = _CHUNK + 8
_NSTREAM = 8
_IW = _CHUNK // _NSTREAM
_NCH = _CHUNK // _LANES


def _tok_body(flat_hbm, cu_hbm, table_hbm, out_hbm, cu_v, ft_v, tok_v, val_v, sem):
    wid = lax.axis_index("s") * _NC + lax.axis_index("c")
    b = wid // 2
    h = wid % 2
    l0 = h * _CHUNK

    pltpu.sync_copy(cu_hbm, cu_v)

    lane = lax.iota(jnp.int32, _LANES)
    b_splat = jnp.full((_LANES,), b, jnp.int32)
    start = jnp.max(plsc.load_gather(cu_v, [b_splat]))
    end_splat = plsc.load_gather(cu_v, [b_splat + 1])
    end = jnp.max(end_splat)

    off_raw = start + l0
    n_valid = jnp.clip(end - off_raw, 0, _CHUNK)

    off = pl.multiple_of(jnp.minimum(off_raw & -8, _TOTAL - _STAGE), 8)
    base = off_raw - off

    @pl.when(n_valid > 0)
    def _stage():
        pltpu.sync_copy(flat_hbm.at[pl.ds(off, _STAGE)], ft_v)

    copies = []
    for r in range(_NSTREAM):
        fire = n_valid > r * _IW

        @pl.when(fire)
        def _prep_and_fire():
            @plsc.parallel_loop(r * (_IW // _LANES), (r + 1) * (_IW // _LANES),
                                unroll=4)
            def prep(j):
                lpos = j * _LANES + lane
                lidx = jnp.clip(base + lpos, 0, _STAGE - 1)
                tok = plsc.load_gather(ft_v, [lidx])
                plsc.store_scatter(tok_v, [lpos], tok)

            pltpu.async_copy(
                table_hbm.at[tok_v.at[pl.ds(r * _IW, _IW)]],
                val_v.at[pl.ds(r * _IW, _IW)],
                sem,
            )

        copies.append(fire)
    for r, fire in enumerate(copies):
        @pl.when(fire)
        def _drain():
            pltpu.make_async_copy(
                table_hbm.at[tok_v.at[pl.ds(r * _IW, _IW)]],
                val_v.at[pl.ds(r * _IW, _IW)],
                sem,
            ).wait()

    jz = n_valid // _LANES

    @pl.when(jz < _NCH)
    def _mixed():
        lpos = jz * _LANES + lane
        val = plsc.load_gather(val_v, [lpos])
        keep = (off_raw + lpos) < end_splat
        plsc.store_scatter(val_v, [lpos], jnp.where(keep, val, 0.0))

    zeros = jnp.zeros((_LANES,), jnp.float32)

    def zfill(j, c):
        plsc.store_scatter(val_v, [j * _LANES + lane], zeros)
        return c

    lax.fori_loop(jz + 1, _NCH, zfill, 0)

    pltpu.sync_copy(val_v, out_hbm.at[b, h])


def kernel(flat_tokens, cu_seqlens, lookup_table):
    mesh = plsc.VectorSubcoreMesh(core_axis_name="c", subcore_axis_name="s")
    out = pl.kernel(
        _tok_body,
        out_type=jax.ShapeDtypeStruct((_B, 2, _CHUNK), jnp.float32),
        mesh=mesh,
        compiler_params=pltpu.CompilerParams(
            needs_layout_passes=False, use_tc_tiling_on_sc=False
        ),
        scratch_types=[
            pltpu.VMEM((_B + 1,), jnp.int32),
            pltpu.VMEM((_STAGE,), jnp.int32),
            pltpu.VMEM((_CHUNK,), jnp.int32),
            pltpu.VMEM((_CHUNK,), jnp.float32),
            pltpu.SemaphoreType.DMA,
        ],
    )(flat_tokens, cu_seqlens.astype(jnp.int32), lookup_table)
    return out.reshape(_B, _L)
